# baseline (speedup 1.0000x reference)
import functools

import jax
import jax.numpy as jnp
from jax import lax
from jax.experimental import pallas as pl
from jax.experimental.pallas import tpu as pltpu
from jax.experimental.pallas import tpu_sc as plsc

NV = 1000
NF = 26
D = 64
B = 16384
ROW_STRIDE = NV + 1
FEAT_STRIDE = NF * (NV + 1)


_TC_BLK = 2048


def _idx_body(x_ref, i_ref, s_ref, xt_ref, ipos_ref):
    I = i_ref[...]
    X = x_ref[...]
    S = s_ref[...]
    mx = jnp.max(I, axis=1, keepdims=True)
    j = lax.broadcasted_iota(jnp.int32, I.shape, 1)
    ipos = jnp.min(jnp.where(I == mx, j, NF), axis=1)
    xt = ipos[:, None] * FEAT_STRIDE + j * ROW_STRIDE + X * S + (1 - S) * NV
    xt_ref[...] = xt
    ipos_ref[...] = ipos


def _tc_index(X, I, S):
    return pl.pallas_call(
        _idx_body,
        grid=(B // _TC_BLK,),
        in_specs=[
            pl.BlockSpec((_TC_BLK, NF), lambda i: (i, 0)),
            pl.BlockSpec((_TC_BLK, NF), lambda i: (i, 0)),
            pl.BlockSpec((_TC_BLK, NF), lambda i: (i, 0)),
        ],
        out_specs=[
            pl.BlockSpec((_TC_BLK, NF), lambda i: (i, 0)),
            pl.BlockSpec((_TC_BLK,), lambda i: (i,)),
        ],
        out_shape=[
            jax.ShapeDtypeStruct((B, NF), jnp.int32),
            jax.ShapeDtypeStruct((B,), jnp.int32),
        ],
    )(X, I, S)


_TR_BLK = 16384
_NROWS = (NV + 1) * NF * NF


def _tr_body(wt_ref, o_ref):
    r = lax.broadcasted_iota(jnp.int32, (D, D), 0)
    c = lax.broadcasted_iota(jnp.int32, (D, D), 1)
    eye = (r == c).astype(jnp.float32)
    o_ref[...] = lax.dot_general(
        wt_ref[...], eye, (((0,), (0,)), ((), ())),
        preferred_element_type=jnp.float32)


def _tc_transpose(wt):
    grid_n = (_NROWS + _TR_BLK - 1) // _TR_BLK
    return pl.pallas_call(
        _tr_body,
        grid=(grid_n,),
        in_specs=[pl.BlockSpec((D, _TR_BLK), lambda i: (0, i))],
        out_specs=pl.BlockSpec((_TR_BLK, D), lambda i: (i, 0)),
        out_shape=jax.ShapeDtypeStruct((_NROWS, D), jnp.float32),
    )(wt)


_NW = 32
_RPT = B // _NW
_CB = 16
_NCHUNK = _RPT // _CB
_IDX = _CB * NF
_GSPLIT = 4
_GN = _IDX // _GSPLIT
_NBUF = 3


def _sc_body(xt_hbm, ipos_hbm, w_hbm, b_hbm, out_hbm,
             xt_t, ipos_t, rows_v, bias_v, out_v, sem_w, sem_b, sem_o):
    c = lax.axis_index("c")
    s = lax.axis_index("s")
    wid = s * 2 + c
    tbase = wid * _RPT

    pltpu.sync_copy(xt_hbm.at[pl.ds(tbase * NF, _RPT * NF)], xt_t)
    pltpu.sync_copy(ipos_hbm.at[pl.ds(tbase, _RPT)], ipos_t)

    def issue(g, pb):
        for i in range(_GSPLIT):
            pltpu.async_copy(
                w_hbm.at[xt_t.at[pl.ds(g * _IDX + i * _GN, _GN)]],
                rows_v.at[pb].at[pl.ds(i * _GN, _GN)], sem_w[pb])
        pltpu.async_copy(b_hbm.at[ipos_t.at[pl.ds(g * _CB, _CB)]],
                         bias_v.at[pb], sem_b[pb])

    def wait_in(g, pb):
        for i in range(_GSPLIT):
            pltpu.make_async_copy(
                w_hbm.at[xt_t.at[pl.ds(g * _IDX + i * _GN, _GN)]],
                rows_v.at[pb].at[pl.ds(i * _GN, _GN)], sem_w[pb]).wait()
        pltpu.make_async_copy(b_hbm.at[ipos_t.at[pl.ds(g * _CB, _CB)]],
                              bias_v.at[pb], sem_b[pb]).wait()

    def out_desc(g, pb):
        return pltpu.make_async_copy(
            out_v.at[pb], out_hbm.at[pl.ds(tbase + g * _CB, _CB), :],
            sem_o[pb])

    def finish(g, pb):
        wait_in(g, pb)

        @pl.when(g >= _NBUF)
        def _():
            out_desc(g - _NBUF, pb).wait()

        def row(r, carry2):
            p0 = r * NF
            acc = [rows_v[pb, p0, pl.ds(k * 16, 16)] for k in range(D // 16)]
            for jj in range(1, NF):
                for k in range(D // 16):
                    acc[k] = acc[k] + rows_v[pb, p0 + jj, pl.ds(k * 16, 16)]
            for k in range(D // 16):
                out_v[pb, r, pl.ds(k * 16, 16)] = (
                    acc[k] * (1.0 / NF) + bias_v[pb, r, pl.ds(k * 16, 16)])
            return carry2

        lax.fori_loop(0, _CB, row, 0)
        out_desc(g, pb).start()

    issue(jnp.int32(0), 0)
    issue(jnp.int32(1), 1)

    def body(t, carry):
        g0 = _NBUF * t
        for u in range(_NBUF):
            issue(g0 + u + 2, (u + 2) % _NBUF)
            finish(g0 + u, u)
        return carry

    lax.fori_loop(0, (_NCHUNK - 2) // _NBUF, body, 0)
    finish(jnp.int32(_NCHUNK - 2), (_NCHUNK - 2) % _NBUF)
    finish(jnp.int32(_NCHUNK - 1), (_NCHUNK - 1) % _NBUF)

    for g in range(_NCHUNK - _NBUF, _NCHUNK):
        out_desc(jnp.int32(g), g % _NBUF).wait()


@functools.partial(
    pl.kernel,
    out_type=jax.ShapeDtypeStruct((B, D), jnp.float32),
    mesh=plsc.VectorSubcoreMesh(core_axis_name="c", subcore_axis_name="s"),
    compiler_params=pltpu.CompilerParams(use_tc_tiling_on_sc=False),
    scratch_types=[
        pltpu.VMEM((_RPT * NF,), jnp.int32),
        pltpu.VMEM((_RPT,), jnp.int32),
        pltpu.VMEM((_NBUF, _IDX, D), jnp.float32),
        pltpu.VMEM((_NBUF, _CB, D), jnp.float32),
        pltpu.VMEM((_NBUF, _CB, D), jnp.float32),
        tuple(pltpu.SemaphoreType.DMA for _ in range(_NBUF)),
        tuple(pltpu.SemaphoreType.DMA for _ in range(_NBUF)),
        tuple(pltpu.SemaphoreType.DMA for _ in range(_NBUF)),
    ],
)
def _sc_gather(xt_hbm, ipos_hbm, w_hbm, b_hbm, out_hbm,
               xt_t, ipos_t, rows_v, bias_v, out_v, sem_w, sem_b, sem_o):
    _sc_body(xt_hbm, ipos_hbm, w_hbm, b_hbm, out_hbm,
             xt_t, ipos_t, rows_v, bias_v, out_v, sem_w, sem_b, sem_o)


def kernel(X, I, S, weights, bias):
    X = X.astype(jnp.int32)
    S = S.astype(jnp.int32)
    I = I.astype(jnp.float32)
    xt, ipos = _tc_index(X, I, S)
    w_rm = _tc_transpose(weights.T)
    return _sc_gather(xt.reshape(-1), ipos, w_rm, bias)


# speedup vs baseline: 1.0684x; 1.0684x over previous
import functools

import jax
import jax.numpy as jnp
from jax import lax
from jax.experimental import pallas as pl
from jax.experimental.pallas import tpu as pltpu
from jax.experimental.pallas import tpu_sc as plsc

NV = 1000
NF = 26
D = 64
B = 16384
ROW_STRIDE = NV + 1
FEAT_STRIDE = NF * (NV + 1)


_TC_BLK = 2048


def _idx_body(x_ref, i_ref, s_ref, xt_ref, ipos_ref):
    I = i_ref[...]
    X = x_ref[...]
    S = s_ref[...]
    mx = jnp.max(I, axis=1, keepdims=True)
    j = lax.broadcasted_iota(jnp.int32, I.shape, 1)
    ipos = jnp.min(jnp.where(I == mx, j, NF), axis=1)
    xt = ipos[:, None] * FEAT_STRIDE + j * ROW_STRIDE + X * S + (1 - S) * NV
    xt_ref[...] = xt
    ipos_ref[...] = ipos


def _tc_index(X, I, S):
    return pl.pallas_call(
        _idx_body,
        grid=(B // _TC_BLK,),
        in_specs=[
            pl.BlockSpec((_TC_BLK, NF), lambda i: (i, 0)),
            pl.BlockSpec((_TC_BLK, NF), lambda i: (i, 0)),
            pl.BlockSpec((_TC_BLK, NF), lambda i: (i, 0)),
        ],
        out_specs=[
            pl.BlockSpec((_TC_BLK, NF), lambda i: (i, 0)),
            pl.BlockSpec((_TC_BLK,), lambda i: (i,)),
        ],
        out_shape=[
            jax.ShapeDtypeStruct((B, NF), jnp.int32),
            jax.ShapeDtypeStruct((B,), jnp.int32),
        ],
    )(X, I, S)


_NW = 32
_RPT = B // _NW
_CB = 16
_NCHUNK = _RPT // _CB
_IDX = _CB * NF
_GSPLIT = 4
_GN = _IDX // _GSPLIT
_NBUF = 3


def _sc_body(xt_hbm, ipos_hbm, w_hbm, b_hbm, out_hbm,
             xt_t, ipos_t, rows_v, bias_v, out_v, sem_w, sem_b, sem_o):
    c = lax.axis_index("c")
    s = lax.axis_index("s")
    wid = s * 2 + c
    tbase = wid * _RPT

    pltpu.sync_copy(xt_hbm.at[pl.ds(tbase * NF, _RPT * NF)], xt_t)
    pltpu.sync_copy(ipos_hbm.at[pl.ds(tbase, _RPT)], ipos_t)

    def issue(g, pb):
        for i in range(_GSPLIT):
            pltpu.async_copy(
                w_hbm.at[xt_t.at[pl.ds(g * _IDX + i * _GN, _GN)]],
                rows_v.at[pb].at[pl.ds(i * _GN, _GN)], sem_w[pb])
        pltpu.async_copy(b_hbm.at[ipos_t.at[pl.ds(g * _CB, _CB)]],
                         bias_v.at[pb], sem_b[pb])

    def wait_in(g, pb):
        for i in range(_GSPLIT):
            pltpu.make_async_copy(
                w_hbm.at[xt_t.at[pl.ds(g * _IDX + i * _GN, _GN)]],
                rows_v.at[pb].at[pl.ds(i * _GN, _GN)], sem_w[pb]).wait()
        pltpu.make_async_copy(b_hbm.at[ipos_t.at[pl.ds(g * _CB, _CB)]],
                              bias_v.at[pb], sem_b[pb]).wait()

    def out_desc(g, pb):
        return pltpu.make_async_copy(
            out_v.at[pb], out_hbm.at[pl.ds(tbase + g * _CB, _CB), :],
            sem_o[pb])

    def finish(g, pb):
        wait_in(g, pb)

        @pl.when(g >= _NBUF)
        def _():
            out_desc(g - _NBUF, pb).wait()

        def row(r, carry2):
            p0 = r * NF
            acc = [rows_v[pb, p0, pl.ds(k * 16, 16)] for k in range(D // 16)]
            for jj in range(1, NF):
                for k in range(D // 16):
                    acc[k] = acc[k] + rows_v[pb, p0 + jj, pl.ds(k * 16, 16)]
            for k in range(D // 16):
                out_v[pb, r, pl.ds(k * 16, 16)] = (
                    acc[k] * (1.0 / NF) + bias_v[pb, r, pl.ds(k * 16, 16)])
            return carry2

        lax.fori_loop(0, _CB, row, 0)
        out_desc(g, pb).start()

    issue(jnp.int32(0), 0)
    issue(jnp.int32(1), 1)

    def body(t, carry):
        g0 = _NBUF * t
        for u in range(_NBUF):
            issue(g0 + u + 2, (u + 2) % _NBUF)
            finish(g0 + u, u)
        return carry

    lax.fori_loop(0, (_NCHUNK - 2) // _NBUF, body, 0)
    finish(jnp.int32(_NCHUNK - 2), (_NCHUNK - 2) % _NBUF)
    finish(jnp.int32(_NCHUNK - 1), (_NCHUNK - 1) % _NBUF)

    for g in range(_NCHUNK - _NBUF, _NCHUNK):
        out_desc(jnp.int32(g), g % _NBUF).wait()


@functools.partial(
    pl.kernel,
    out_type=jax.ShapeDtypeStruct((B, D), jnp.float32),
    mesh=plsc.VectorSubcoreMesh(core_axis_name="c", subcore_axis_name="s"),
    compiler_params=pltpu.CompilerParams(use_tc_tiling_on_sc=False),
    scratch_types=[
        pltpu.VMEM((_RPT * NF,), jnp.int32),
        pltpu.VMEM((_RPT,), jnp.int32),
        pltpu.VMEM((_NBUF, _IDX, D), jnp.float32),
        pltpu.VMEM((_NBUF, _CB, D), jnp.float32),
        pltpu.VMEM((_NBUF, _CB, D), jnp.float32),
        tuple(pltpu.SemaphoreType.DMA for _ in range(_NBUF)),
        tuple(pltpu.SemaphoreType.DMA for _ in range(_NBUF)),
        tuple(pltpu.SemaphoreType.DMA for _ in range(_NBUF)),
    ],
)
def _sc_gather(xt_hbm, ipos_hbm, w_hbm, b_hbm, out_hbm,
               xt_t, ipos_t, rows_v, bias_v, out_v, sem_w, sem_b, sem_o):
    _sc_body(xt_hbm, ipos_hbm, w_hbm, b_hbm, out_hbm,
             xt_t, ipos_t, rows_v, bias_v, out_v, sem_w, sem_b, sem_o)


def kernel(X, I, S, weights, bias):
    X = X.astype(jnp.int32)
    S = S.astype(jnp.int32)
    I = I.astype(jnp.float32)
    xt, ipos = _tc_index(X, I, S)
    return _sc_gather(xt.reshape(-1), ipos, weights, bias)
